# baseline (speedup 1.0000x reference)
import functools

import jax
import jax.numpy as jnp
from jax import lax
from jax.experimental import pallas as pl
from jax.experimental.pallas import tpu as pltpu
from jax.experimental.pallas import tpu_sc as plsc

SEQ = 8192
D = 64
NC = 2
NS = 16
NW = NC * NS
BPW = SEQ // NW
HALF = BPW // 2
LANES = 16

_mesh = plsc.VectorSubcoreMesh(core_axis_name="c", subcore_axis_name="s")


@functools.partial(
    pl.kernel,
    mesh=_mesh,
    out_type=jax.ShapeDtypeStruct((SEQ, D), jnp.float32),
    compiler_params=pltpu.CompilerParams(use_tc_tiling_on_sc=False),
    scratch_types=[
        pltpu.VMEM((2, HALF), jnp.int32),
        pltpu.VMEM((BPW, D), jnp.float32),
        pltpu.VMEM((BPW, D), jnp.float32),
        pltpu.SemaphoreType.DMA,
    ],
)
def _emb(x_hbm, tok_hbm, pos_hbm, out_hbm, idx_v, rows_v, pos_v, sem):
    wid = lax.axis_index("s") * NC + lax.axis_index("c")
    base = wid * BPW

    pltpu.sync_copy(x_hbm.at[pl.ds(base, HALF)], idx_v.at[0])
    pltpu.sync_copy(x_hbm.at[pl.ds(base + HALF, HALF)], idx_v.at[1])

    cp0 = pltpu.async_copy(tok_hbm.at[idx_v.at[0]], rows_v.at[pl.ds(0, HALF)], sem)
    cp1 = pltpu.async_copy(tok_hbm.at[idx_v.at[1]], rows_v.at[pl.ds(HALF, HALF)], sem)
    pltpu.sync_copy(pos_hbm.at[pl.ds(SEQ - base - BPW, BPW)], pos_v)
    cp0.wait()
    cp1.wait()

    def body(j, carry):
        r = BPW - 1 - j
        for c in range(D // LANES):
            rows_v[j, pl.ds(c * LANES, LANES)] += pos_v[r, pl.ds(c * LANES, LANES)]
        return carry

    lax.fori_loop(0, BPW, body, 0)

    pltpu.sync_copy(rows_v, out_hbm.at[pl.ds(base, BPW)])


def kernel(x, token_table, pos_table):
    return _emb(x.astype(jnp.int32), token_table, pos_table)
